# baseline (speedup 1.0000x reference)
import functools

import jax
import jax.numpy as jnp
from jax import lax
from jax.experimental import pallas as pl
from jax.experimental.pallas import tpu as pltpu
from jax.experimental.pallas import tpu_sc as plsc

N = 10000
M = 10000
E = 320000
FEAT = 128
HID = 64
OUT = 10

NC, NS, L = 2, 16, 16
NW = NC * NS
P = 10240
CH = 64
EW = 10240
GPW = EW // CH
EP = EW * NW
KCH = 128
EP2 = EP + KCH
SL = P // NS
SLW = P // NW

_MESH = plsc.VectorSubcoreMesh(core_axis_name="c", subcore_axis_name="s")
_SC_PARAMS = pltpu.CompilerParams(needs_layout_passes=False, use_tc_tiling_on_sc=False)


def _leaky(x, slope):
    return jnp.where(x > 0, x, slope * x)


def _wid():
    return lax.axis_index("s") * NC + lax.axis_index("c")


def _zero16():
    return jnp.zeros((L,), jnp.float32)


def _t1_body(x_ref, ea_ref, w_ref, att_ref, xl_ref, a_ref, b_ref):
    w = w_ref[...]
    xl = jnp.dot(x_ref[...], w, preferred_element_type=jnp.float32)
    el = jnp.dot(ea_ref[...], w, preferred_element_type=jnp.float32)
    xl_ref[...] = xl
    a_ref[...] = jnp.dot(xl, att_ref[0:FEAT, :], preferred_element_type=jnp.float32)
    b_ref[...] = jnp.dot(el, att_ref[FEAT:2 * FEAT, :], preferred_element_type=jnp.float32)


_t1 = pl.pallas_call(
    _t1_body,
    out_shape=[
        jax.ShapeDtypeStruct((P, FEAT), jnp.float32),
        jax.ShapeDtypeStruct((P, 1), jnp.float32),
        jax.ShapeDtypeStruct((P, 1), jnp.float32),
    ],
)


def _gnorm(y, gw, gb, gms, mask):
    cnt = jnp.float32(N)
    mean = jnp.sum(jnp.where(mask, y, 0.0), axis=0, keepdims=True) / cnt
    out = y - mean * gms
    om = jnp.where(mask, out, 0.0)
    var = jnp.sum(om * om, axis=0, keepdims=True) / cnt
    return gw * out / jnp.sqrt(var + 1e-5) + gb


def _t2_body(o0_ref, o1_ref, bias_ref, gw_ref, gb_ref, gms_ref, fw_ref, fb_ref,
             ea_ref, w2_ref, att_ref, xl_ref, a_ref, b_ref, ofc_ref):
    y = o0_ref[...] + o1_ref[...] + bias_ref[...]
    mask = lax.broadcasted_iota(jnp.int32, (P, FEAT), 0) < N
    h = _leaky(_gnorm(y, gw_ref[...], gb_ref[...], gms_ref[...], mask), 0.01)
    ofc_ref[...] = _leaky(
        jnp.dot(h, fw_ref[...], preferred_element_type=jnp.float32) + fb_ref[...], 0.01)
    w2 = w2_ref[...]
    xl = jnp.dot(h, w2, preferred_element_type=jnp.float32)
    el = jnp.dot(ea_ref[...], w2, preferred_element_type=jnp.float32)
    xl_ref[...] = xl
    a_ref[...] = jnp.dot(xl, att_ref[0:FEAT, :], preferred_element_type=jnp.float32)
    b_ref[...] = jnp.dot(el, att_ref[FEAT:2 * FEAT, :], preferred_element_type=jnp.float32)


_t2 = pl.pallas_call(
    _t2_body,
    out_shape=[
        jax.ShapeDtypeStruct((P, FEAT), jnp.float32),
        jax.ShapeDtypeStruct((P, 1), jnp.float32),
        jax.ShapeDtypeStruct((P, 1), jnp.float32),
        jax.ShapeDtypeStruct((P, HID), jnp.float32),
    ],
)


def _t3_body(o0_ref, o1_ref, bias_ref, gw_ref, gb_ref, gms_ref, fw_ref, fb_ref,
             ofc_ref, cw_ref, cb_ref, res_ref):
    y = o0_ref[...] + o1_ref[...] + bias_ref[...]
    mask = lax.broadcasted_iota(jnp.int32, (P, FEAT), 0) < N
    h2 = _leaky(_gnorm(y, gw_ref[...], gb_ref[...], gms_ref[...], mask), 0.01)
    out = ofc_ref[...] + _leaky(
        jnp.dot(h2, fw_ref[...], preferred_element_type=jnp.float32) + fb_ref[...], 0.01)
    res_ref[...] = jnp.dot(out, cw_ref[...], preferred_element_type=jnp.float32) + cb_ref[...]


_t3 = pl.pallas_call(
    _t3_body,
    out_shape=jax.ShapeDtypeStruct((P, OUT), jnp.float32),
)


def _make_k1(with_counts):
    out_type = [jax.ShapeDtypeStruct((EP2,), jnp.float32),
                jax.ShapeDtypeStruct((NC * P,), jnp.float32)]
    scratch = [
        pltpu.VMEM((P,), jnp.float32),
        pltpu.VMEM((P,), jnp.float32),
        pltpu.VMEM((KCH,), jnp.int32),
        pltpu.VMEM((KCH,), jnp.int32),
        pltpu.VMEM((KCH,), jnp.int32),
        pltpu.VMEM((KCH,), jnp.int32),
        pltpu.VMEM((KCH,), jnp.int32),
        pltpu.VMEM((KCH,), jnp.int32),
        pltpu.VMEM((KCH,), jnp.int32),
        pltpu.VMEM((KCH,), jnp.int32),
        pltpu.VMEM((KCH,), jnp.float32),
        pltpu.VMEM((KCH,), jnp.float32),
        pltpu.VMEM((SL,), jnp.float32),
        pltpu.VMEM_SHARED((P,), jnp.float32),
        pltpu.SemaphoreType.DMA,
        pltpu.SemaphoreType.DMA,
        pltpu.SemaphoreType.DMA,
        pltpu.SemaphoreType.DMA,
    ]
    if with_counts:
        out_type += [jax.ShapeDtypeStruct((NC * P,), jnp.float32),
                     jax.ShapeDtypeStruct((NC * P,), jnp.float32)]
        scratch += [
            pltpu.VMEM((KCH,), jnp.float32),
            pltpu.VMEM_SHARED((P,), jnp.float32),
            pltpu.VMEM_SHARED((P,), jnp.float32),
        ]

    def body(row_h, col_h, an_h, be_h, *rest):
        if with_counts:
            (eexp_h, dnm_h, bc_h, dc_h,
             an_v, be_v, rv0, rv1, cv0, cv1, sr0, sr1, sc0, sc1, ev0, ev1,
             z_v, dnm_s, sem_i0, sem_i1, sem_s0, sem_s1, one_v, b_s, d_s) = rest
        else:
            (eexp_h, dnm_h,
             an_v, be_v, rv0, rv1, cv0, cv1, sr0, sr1, sc0, sc1, ev0, ev1,
             z_v, dnm_s, sem_i0, sem_i1, sem_s0, sem_s1) = rest
        c = lax.axis_index("c")
        s = lax.axis_index("s")
        wid = _wid()
        row_b = (rv0, rv1)
        col_b = (cv0, cv1)
        sr_b = (sr0, sr1)
        sc_b = (sc0, sc1)
        e_b = (ev0, ev1)
        sem_i = (sem_i0, sem_i1)
        sem_s = (sem_s0, sem_s1)

        def zb(i, _):
            z_v[pl.ds(pl.multiple_of(i * L, L), L)] = _zero16()
            return 0
        lax.fori_loop(0, SL // L, zb, 0)
        soff = pl.multiple_of(s * SL, SL)
        pltpu.sync_copy(z_v, dnm_s.at[pl.ds(soff, SL)])
        if with_counts:
            pltpu.sync_copy(z_v, b_s.at[pl.ds(soff, SL)])
            pltpu.sync_copy(z_v, d_s.at[pl.ds(soff, SL)])
            for j in range(KCH // L):
                one_v[pl.ds(j * L, L)] = jnp.full((L,), 1.0, jnp.float32)
        pltpu.sync_copy(an_h, an_v)
        pltpu.sync_copy(be_h, be_v)
        plsc.subcore_barrier()

        base = pl.multiple_of(wid * EW, EW)

        def off_of(g):
            return pl.multiple_of(base + g * KCH, KCH)

        def issue_idx(g, b):
            off = off_of(g)
            pltpu.async_copy(row_h.at[pl.ds(off, KCH)], row_b[b], sem_i[b])
            pltpu.async_copy(col_h.at[pl.ds(off, KCH)], col_b[b], sem_i[b])

        def drain_idx(g, b):
            off = off_of(g)
            pltpu.make_async_copy(row_h.at[pl.ds(off, KCH)], row_b[b], sem_i[b]).wait()
            pltpu.make_async_copy(col_h.at[pl.ds(off, KCH)], col_b[b], sem_i[b]).wait()

        def half(g, b):
            drain_idx(g, b)
            for j in range(KCH // L):
                sl = pl.ds(j * L, L)
                rr = row_b[b][sl]
                cc = col_b[b][sl]
                av = plsc.load_gather(an_v, [rr])
                bv = plsc.load_gather(be_v, [cc])
                e_b[b][sl] = jnp.exp(_leaky(av + bv, 0.2))
                sr_b[b][sl] = rr
                sc_b[b][sl] = cc
            pltpu.sync_copy(e_b[b], eexp_h.at[pl.ds(off_of(g), KCH)])
            pltpu.sync_copy(e_b[b], dnm_s.at[sc_b[b]], add=True)
            if with_counts:
                pltpu.sync_copy(one_v, b_s.at[sc_b[b]], add=True)
                pltpu.sync_copy(one_v, d_s.at[sr_b[b]], add=True)

        issue_idx(0, 0)

        def pair(t, _):
            ga = 2 * t
            issue_idx(ga + 1, 1)
            half(ga, 0)
            issue_idx(ga + 2, 0)
            half(ga + 1, 1)
            return 0
        lax.fori_loop(0, (EW // KCH) // 2, pair, 0)
        drain_idx(EW // KCH, 0)

        plsc.subcore_barrier()
        doff = pl.multiple_of(c * P + s * SL, SL)
        pltpu.sync_copy(dnm_s.at[pl.ds(soff, SL)], dnm_h.at[pl.ds(doff, SL)])
        if with_counts:
            pltpu.sync_copy(b_s.at[pl.ds(soff, SL)], bc_h.at[pl.ds(doff, SL)])
            pltpu.sync_copy(d_s.at[pl.ds(soff, SL)], dc_h.at[pl.ds(doff, SL)])

    return pl.kernel(body, out_type=out_type, mesh=_MESH, scratch_types=scratch,
                     compiler_params=_SC_PARAMS)


_k1_counts = _make_k1(True)
_k1_plain = _make_k1(False)


def _make_k2(full):
    n_out = 4 if full else 2
    out_type = [jax.ShapeDtypeStruct((P,), jnp.float32)] * n_out
    scratch = [pltpu.VMEM((SLW,), jnp.float32) for _ in range(3)]

    def body(*args):
        if full:
            (dnm_h, bc_h, dc_h, dnminv_h, ubd_h, binv_h, dinv_h, t0, t1, t2) = args
        else:
            (dnm_h, binv_in_h, dnminv_h, ubd_h, t0, t1, t2) = args
        wid = _wid()
        off = pl.multiple_of(wid * SLW, SLW)

        pltpu.sync_copy(dnm_h.at[pl.ds(off, SLW)], t0)
        pltpu.sync_copy(dnm_h.at[pl.ds(P + off, SLW)], t1)
        for j in range(SLW // L):
            sl = pl.ds(j * L, L)
            t0[sl] = 1.0 / (t0[sl] + t1[sl] + 1e-16)
        pltpu.sync_copy(t0, dnminv_h.at[pl.ds(off, SLW)])

        if full:
            pltpu.sync_copy(bc_h.at[pl.ds(off, SLW)], t1)
            pltpu.sync_copy(bc_h.at[pl.ds(P + off, SLW)], t2)
            for j in range(SLW // L):
                sl = pl.ds(j * L, L)
                b = t1[sl] + t2[sl]
                bi = jnp.where(b > 0, 1.0 / jnp.where(b > 0, b, 1.0), 0.0)
                t1[sl] = bi
                t2[sl] = bi * t0[sl]
            pltpu.sync_copy(t1, binv_h.at[pl.ds(off, SLW)])
            pltpu.sync_copy(t2, ubd_h.at[pl.ds(off, SLW)])

            pltpu.sync_copy(dc_h.at[pl.ds(off, SLW)], t1)
            pltpu.sync_copy(dc_h.at[pl.ds(P + off, SLW)], t2)
            for j in range(SLW // L):
                sl = pl.ds(j * L, L)
                d = t1[sl] + t2[sl]
                t1[sl] = jnp.where(d > 0, 1.0 / jnp.where(d > 0, d, 1.0), 0.0)
            pltpu.sync_copy(t1, dinv_h.at[pl.ds(off, SLW)])
        else:
            pltpu.sync_copy(binv_in_h.at[pl.ds(off, SLW)], t1)
            for j in range(SLW // L):
                sl = pl.ds(j * L, L)
                t2[sl] = t1[sl] * t0[sl]
            pltpu.sync_copy(t2, ubd_h.at[pl.ds(off, SLW)])

    return pl.kernel(body, out_type=out_type, mesh=_MESH, scratch_types=scratch,
                     compiler_params=_SC_PARAMS)


_k2_full = _make_k2(True)
_k2_small = _make_k2(False)


def _make_k3(stage1):
    out_type = [jax.ShapeDtypeStruct((NC * P, FEAT), jnp.float32)]
    if stage1:
        out_type.append(jax.ShapeDtypeStruct((EP2,), jnp.float32))
    scratch = [
        pltpu.VMEM((P,), jnp.float32),
        pltpu.VMEM((CH,), jnp.int32),
        pltpu.VMEM((CH,), jnp.int32),
        pltpu.VMEM((CH,), jnp.int32),
        pltpu.VMEM((CH,), jnp.int32),
        pltpu.VMEM((CH,), jnp.int32),
        pltpu.VMEM((CH,), jnp.int32),
        pltpu.VMEM((CH,), jnp.float32),
        pltpu.VMEM((CH,), jnp.float32),
        pltpu.VMEM((CH,), jnp.float32),
        pltpu.VMEM((CH,), jnp.float32),
        pltpu.VMEM((CH, FEAT), jnp.float32),
        pltpu.VMEM((CH, FEAT), jnp.float32),
        pltpu.VMEM_SHARED((P, FEAT), jnp.float32),
        pltpu.SemaphoreType.DMA,
        pltpu.SemaphoreType.DMA,
        pltpu.SemaphoreType.DMA,
        pltpu.SemaphoreType.DMA,
        pltpu.SemaphoreType.DMA,
        pltpu.SemaphoreType.DMA,
        pltpu.SemaphoreType.DMA,
    ]
    if stage1:
        scratch.insert(1, pltpu.VMEM((P,), jnp.float32))
        scratch[12:12] = [
            pltpu.VMEM((CH,), jnp.float32),
            pltpu.VMEM((CH,), jnp.float32),
        ]
    else:
        scratch[7:7] = [
            pltpu.VMEM((CH,), jnp.int32),
            pltpu.VMEM((CH,), jnp.int32),
        ]
        scratch[15:15] = [
            pltpu.VMEM((CH, FEAT), jnp.float32),
            pltpu.VMEM((CH, FEAT), jnp.float32),
        ]

    def body(row_h, col_h, e_h, *rest):
        if stage1:
            (ubd_h, dnminv_h, src_h, dst_h, enorm_h,
             u_v, dnm_v, rv0, rv1, cv0, cv1, si0, si1, ev0, ev1, wv0, wv1,
             en0, en1, rs0, rs1, dst_s,
             sem_i0, sem_i1, sem_g0, sem_g1, sem_s0, sem_s1, sem_e) = rest
        else:
            (dinv_h, src_h, dst_h,
             u_v, rv0, rv1, cv0, cv1, si0, si1, sj0, sj1, ev0, ev1, wv0, wv1,
             rs0, rs1, rq0, rq1, dst_s,
             sem_i0, sem_i1, sem_g0, sem_g1, sem_s0, sem_s1, sem_e) = rest
        c = lax.axis_index("c")
        s = lax.axis_index("s")
        wid = _wid()
        row_b = (rv0, rv1)
        col_b = (cv0, cv1)
        si_b = (si0, si1)
        e_b = (ev0, ev1)
        w_b = (wv0, wv1)
        if stage1:
            en_b = (en0, en1)
        else:
            si2_b = (sj0, sj1)
            rows2_b = (rq0, rq1)
        rows_b = (rs0, rs1)
        sem_i = (sem_i0, sem_i1)
        sem_g = (sem_g0, sem_g1)
        sem_s = (sem_s0, sem_s1)

        def zrow(i, _):
            for j in range(FEAT // L):
                rs0[i, pl.ds(j * L, L)] = _zero16()
            return 0
        lax.fori_loop(0, CH, zrow, 0)
        soff = pl.multiple_of(s * SL, SL)
        for k in range(SL // CH):
            pltpu.sync_copy(rs0, dst_s.at[pl.ds(soff + k * CH, CH)])

        if stage1:
            pltpu.sync_copy(ubd_h, u_v)
            pltpu.sync_copy(dnminv_h, dnm_v)
        else:
            pltpu.sync_copy(dinv_h, u_v)
        plsc.subcore_barrier()

        base = pl.multiple_of(wid * EW, EW)

        def off_of(g):
            return pl.multiple_of(base + g * CH, CH)

        def issue_idx(g, b):
            off = off_of(g)
            pltpu.async_copy(row_h.at[pl.ds(off, CH)], row_b[b], sem_i[b])
            pltpu.async_copy(col_h.at[pl.ds(off, CH)], col_b[b], sem_i[b])
            pltpu.async_copy(e_h.at[pl.ds(off, CH)], e_b[b], sem_i[b])

        def drain_idx(g, b):
            off = off_of(g)
            pltpu.make_async_copy(row_h.at[pl.ds(off, CH)], row_b[b], sem_i[b]).wait()
            pltpu.make_async_copy(col_h.at[pl.ds(off, CH)], col_b[b], sem_i[b]).wait()
            pltpu.make_async_copy(e_h.at[pl.ds(off, CH)], e_b[b], sem_i[b]).wait()

        def gather_of(b):
            if stage1:
                return [pltpu.async_copy(src_h.at[row_b[b]], rows_b[b], sem_g[b])]
            for j in range(CH // L):
                sl = pl.ds(j * L, L)
                si2_b[b][sl] = col_b[b][sl] + P
            return [pltpu.async_copy(src_h.at[col_b[b]], rows_b[b], sem_g[b]),
                    pltpu.async_copy(src_h.at[si2_b[b]], rows2_b[b], sem_g[b])]

        def weights_of(g, b):
            dst_idx = col_b[b] if stage1 else row_b[b]
            for j in range(CH // L):
                sl = pl.ds(j * L, L)
                ii = dst_idx[sl]
                uu = plsc.load_gather(u_v, [ii])
                ee = e_b[b][sl]
                w_b[b][sl] = ee * uu
                si_b[b][sl] = ii
                if stage1:
                    dn = plsc.load_gather(dnm_v, [ii])
                    en_b[b][sl] = ee * dn
            if stage1:
                return pltpu.async_copy(
                    en_b[b], enorm_h.at[pl.ds(off_of(g), CH)], sem_e)
            return None

        def scale_of(b):
            def scale(b2, _):
                wvec = w_b[b][pl.ds(pl.multiple_of(b2 * L, L), L)]
                for i in range(L):
                    r = b2 * L + i
                    w = wvec[i]
                    for j in range(FEAT // L):
                        sl = pl.ds(j * L, L)
                        if stage1:
                            rows_b[b][r, sl] = rows_b[b][r, sl] * w
                        else:
                            rows_b[b][r, sl] = (
                                rows_b[b][r, sl] + rows2_b[b][r, sl]) * w
                return 0
            lax.fori_loop(0, CH // L, scale, 0)

        issue_idx(0, 0)

        def pair(t, _):
            ga = 2 * t
            issue_idx(ga + 1, 1)
            drain_idx(ga, 0)
            gcp0 = gather_of(0)
            ecp0 = weights_of(ga, 0)
            drain_idx(ga + 1, 1)
            gcp1 = gather_of(1)
            ecp1 = weights_of(ga + 1, 1)
            for cp in gcp0:
                cp.wait()
            issue_idx(ga + 2, 0)
            scale_of(0)
            scp0 = pltpu.async_copy(rows_b[0], dst_s.at[si_b[0]], sem_s[0], add=True)
            for cp in gcp1:
                cp.wait()
            scale_of(1)
            scp0.wait()
            scp1 = pltpu.async_copy(rows_b[1], dst_s.at[si_b[1]], sem_s[1], add=True)
            if stage1:
                ecp0.wait()
            scp1.wait()
            if stage1:
                ecp1.wait()
            return 0
        lax.fori_loop(0, GPW // 2, pair, 0)
        drain_idx(GPW, 0)

        plsc.subcore_barrier()
        for k in range(SL // CH):
            pltpu.sync_copy(
                dst_s.at[pl.ds(soff + k * CH, CH)],
                dst_h.at[pl.ds(pl.multiple_of(c * P + s * SL + k * CH, CH), CH)])

    return pl.kernel(body, out_type=out_type, mesh=_MESH, scratch_types=scratch,
                     compiler_params=_SC_PARAMS)


_k3_stage1 = _make_k3(True)
_k3_stage2 = _make_k3(False)


def kernel(x, edge_index, edge_attr, W1, att1, b1, W2, att2, b2,
           gn1_w, gn1_b, gn1_ms, gn2_w, gn2_b, gn2_ms,
           fc1_w, fc1_b, fc2_w, fc2_b, cls_w, cls_b):
    xp = jnp.pad(x, ((0, P - N), (0, 0)))
    eap = jnp.pad(edge_attr, ((0, P - M), (0, 0)))
    pad_idx = (jnp.arange(EP2 - E, dtype=jnp.int32) % (P - N)) + N
    rowp = jnp.concatenate([edge_index[0], pad_idx])
    colp = jnp.concatenate([edge_index[1], pad_idx])

    att1c = att1.reshape(2 * FEAT, 1)
    att2c = att2.reshape(2 * FEAT, 1)

    xl1, a1, b1v = _t1(xp, eap, W1, att1c)
    eexp1, dnm_p, bc_p, dc_p = _k1_counts(rowp, colp, a1.reshape(P), b1v.reshape(P))
    dnminv1, ubd1, binv, dinv = _k2_full(dnm_p, bc_p, dc_p)
    eout_p, enorm1 = _k3_stage1(rowp, colp, eexp1, ubd1, dnminv1, xl1)
    (out_p1,) = _k3_stage2(rowp, colp, enorm1, dinv, eout_p)

    xl2, a2, b2v, ofc1 = _t2(
        out_p1[:P], out_p1[P:], b1.reshape(1, FEAT),
        gn1_w.reshape(1, FEAT), gn1_b.reshape(1, FEAT), gn1_ms.reshape(1, FEAT),
        fc1_w, fc1_b.reshape(1, HID), eap, W2, att2c)

    eexp2, dnm_p2 = _k1_plain(rowp, colp, a2.reshape(P), b2v.reshape(P))
    dnminv2, ubd2 = _k2_small(dnm_p2, binv)
    eout_p2, enorm2 = _k3_stage1(rowp, colp, eexp2, ubd2, dnminv2, xl2)
    (out_p2,) = _k3_stage2(rowp, colp, enorm2, dinv, eout_p2)

    res = _t3(
        out_p2[:P], out_p2[P:], b2.reshape(1, FEAT),
        gn2_w.reshape(1, FEAT), gn2_b.reshape(1, FEAT), gn2_ms.reshape(1, FEAT),
        fc2_w, fc2_b.reshape(1, HID), ofc1, cls_w, cls_b.reshape(1, OUT))
    return res[:N]


# speedup vs baseline: 1.5256x; 1.5256x over previous
import functools

import jax
import jax.numpy as jnp
from jax import lax
from jax.experimental import pallas as pl
from jax.experimental.pallas import tpu as pltpu
from jax.experimental.pallas import tpu_sc as plsc

N = 10000
M = 10000
E = 320000
FEAT = 128
HID = 64
OUT = 10

NC, NS, L = 2, 16, 16
NW = NC * NS
P = 10240
CH = 64
EW = 10240
GPW = EW // CH
EP = EW * NW
KCH = 128
EP2 = EP + KCH
SL = P // NS
SLW = P // NW

_MESH = plsc.VectorSubcoreMesh(core_axis_name="c", subcore_axis_name="s")
_SC_PARAMS = pltpu.CompilerParams(needs_layout_passes=False, use_tc_tiling_on_sc=False)


def _leaky(x, slope):
    return jnp.where(x > 0, x, slope * x)


def _wid():
    return lax.axis_index("s") * NC + lax.axis_index("c")


def _zero16():
    return jnp.zeros((L,), jnp.float32)


def _t1_body(x_ref, ea_ref, w_ref, att_ref, xl_ref, a_ref, b_ref):
    w = w_ref[...]
    xl = jnp.dot(x_ref[...], w, preferred_element_type=jnp.float32)
    el = jnp.dot(ea_ref[...], w, preferred_element_type=jnp.float32)
    xl_ref[...] = xl
    a_ref[...] = jnp.dot(xl, att_ref[0:FEAT, :], preferred_element_type=jnp.float32)
    b_ref[...] = jnp.dot(el, att_ref[FEAT:2 * FEAT, :], preferred_element_type=jnp.float32)


_t1 = pl.pallas_call(
    _t1_body,
    out_shape=[
        jax.ShapeDtypeStruct((P, FEAT), jnp.float32),
        jax.ShapeDtypeStruct((P, 1), jnp.float32),
        jax.ShapeDtypeStruct((P, 1), jnp.float32),
    ],
)


def _gnorm(y, gw, gb, gms, mask):
    cnt = jnp.float32(N)
    mean = jnp.sum(jnp.where(mask, y, 0.0), axis=0, keepdims=True) / cnt
    out = y - mean * gms
    om = jnp.where(mask, out, 0.0)
    var = jnp.sum(om * om, axis=0, keepdims=True) / cnt
    return gw * out / jnp.sqrt(var + 1e-5) + gb


def _t2_body(o0_ref, o1_ref, bias_ref, gw_ref, gb_ref, gms_ref, fw_ref, fb_ref,
             ea_ref, w2_ref, att_ref, xl_ref, a_ref, b_ref, ofc_ref):
    y = o0_ref[...] + o1_ref[...] + bias_ref[...]
    mask = lax.broadcasted_iota(jnp.int32, (P, FEAT), 0) < N
    h = _leaky(_gnorm(y, gw_ref[...], gb_ref[...], gms_ref[...], mask), 0.01)
    ofc_ref[...] = _leaky(
        jnp.dot(h, fw_ref[...], preferred_element_type=jnp.float32) + fb_ref[...], 0.01)
    w2 = w2_ref[...]
    xl = jnp.dot(h, w2, preferred_element_type=jnp.float32)
    el = jnp.dot(ea_ref[...], w2, preferred_element_type=jnp.float32)
    xl_ref[...] = xl
    a_ref[...] = jnp.dot(xl, att_ref[0:FEAT, :], preferred_element_type=jnp.float32)
    b_ref[...] = jnp.dot(el, att_ref[FEAT:2 * FEAT, :], preferred_element_type=jnp.float32)


_t2 = pl.pallas_call(
    _t2_body,
    out_shape=[
        jax.ShapeDtypeStruct((P, FEAT), jnp.float32),
        jax.ShapeDtypeStruct((P, 1), jnp.float32),
        jax.ShapeDtypeStruct((P, 1), jnp.float32),
        jax.ShapeDtypeStruct((P, HID), jnp.float32),
    ],
)


def _t3_body(o0_ref, o1_ref, bias_ref, gw_ref, gb_ref, gms_ref, fw_ref, fb_ref,
             ofc_ref, cw_ref, cb_ref, res_ref):
    y = o0_ref[...] + o1_ref[...] + bias_ref[...]
    mask = lax.broadcasted_iota(jnp.int32, (P, FEAT), 0) < N
    h2 = _leaky(_gnorm(y, gw_ref[...], gb_ref[...], gms_ref[...], mask), 0.01)
    out = ofc_ref[...] + _leaky(
        jnp.dot(h2, fw_ref[...], preferred_element_type=jnp.float32) + fb_ref[...], 0.01)
    res_ref[...] = jnp.dot(out, cw_ref[...], preferred_element_type=jnp.float32) + cb_ref[...]


_t3 = pl.pallas_call(
    _t3_body,
    out_shape=jax.ShapeDtypeStruct((P, OUT), jnp.float32),
)


def _tadd_body(a_ref, b_ref, o_ref):
    o_ref[...] = a_ref[...] + b_ref[...]


_tadd = pl.pallas_call(
    _tadd_body,
    out_shape=jax.ShapeDtypeStruct((P, FEAT), jnp.float32),
)


def _make_k1(with_counts):
    out_type = [jax.ShapeDtypeStruct((EP2,), jnp.float32),
                jax.ShapeDtypeStruct((NC * P,), jnp.float32)]
    scratch = [
        pltpu.VMEM((P,), jnp.float32),
        pltpu.VMEM((P,), jnp.float32),
        pltpu.VMEM((KCH,), jnp.int32),
        pltpu.VMEM((KCH,), jnp.int32),
        pltpu.VMEM((KCH,), jnp.int32),
        pltpu.VMEM((KCH,), jnp.int32),
        pltpu.VMEM((KCH,), jnp.int32),
        pltpu.VMEM((KCH,), jnp.int32),
        pltpu.VMEM((KCH,), jnp.int32),
        pltpu.VMEM((KCH,), jnp.int32),
        pltpu.VMEM((KCH,), jnp.float32),
        pltpu.VMEM((KCH,), jnp.float32),
        pltpu.VMEM((SL,), jnp.float32),
        pltpu.VMEM_SHARED((P,), jnp.float32),
        pltpu.SemaphoreType.DMA,
        pltpu.SemaphoreType.DMA,
        pltpu.SemaphoreType.DMA,
        pltpu.SemaphoreType.DMA,
    ]
    if with_counts:
        out_type += [jax.ShapeDtypeStruct((NC * P,), jnp.float32),
                     jax.ShapeDtypeStruct((NC * P,), jnp.float32)]
        scratch += [
            pltpu.VMEM((KCH,), jnp.float32),
            pltpu.VMEM_SHARED((P,), jnp.float32),
            pltpu.VMEM_SHARED((P,), jnp.float32),
        ]

    def body(row_h, col_h, an_h, be_h, *rest):
        if with_counts:
            (eexp_h, dnm_h, bc_h, dc_h,
             an_v, be_v, rv0, rv1, cv0, cv1, sr0, sr1, sc0, sc1, ev0, ev1,
             z_v, dnm_s, sem_i0, sem_i1, sem_s0, sem_s1, one_v, b_s, d_s) = rest
        else:
            (eexp_h, dnm_h,
             an_v, be_v, rv0, rv1, cv0, cv1, sr0, sr1, sc0, sc1, ev0, ev1,
             z_v, dnm_s, sem_i0, sem_i1, sem_s0, sem_s1) = rest
        c = lax.axis_index("c")
        s = lax.axis_index("s")
        wid = _wid()
        row_b = (rv0, rv1)
        col_b = (cv0, cv1)
        sr_b = (sr0, sr1)
        sc_b = (sc0, sc1)
        e_b = (ev0, ev1)
        sem_i = (sem_i0, sem_i1)
        sem_s = (sem_s0, sem_s1)

        def zb(i, _):
            z_v[pl.ds(pl.multiple_of(i * L, L), L)] = _zero16()
            return 0
        lax.fori_loop(0, SL // L, zb, 0)
        soff = pl.multiple_of(s * SL, SL)
        pltpu.sync_copy(z_v, dnm_s.at[pl.ds(soff, SL)])
        if with_counts:
            pltpu.sync_copy(z_v, b_s.at[pl.ds(soff, SL)])
            pltpu.sync_copy(z_v, d_s.at[pl.ds(soff, SL)])
            for j in range(KCH // L):
                one_v[pl.ds(j * L, L)] = jnp.full((L,), 1.0, jnp.float32)
        pltpu.sync_copy(an_h, an_v)
        pltpu.sync_copy(be_h, be_v)
        plsc.subcore_barrier()

        base = pl.multiple_of(wid * EW, EW)

        def off_of(g):
            return pl.multiple_of(base + g * KCH, KCH)

        def issue_idx(g, b):
            off = off_of(g)
            pltpu.async_copy(row_h.at[pl.ds(off, KCH)], row_b[b], sem_i[b])
            pltpu.async_copy(col_h.at[pl.ds(off, KCH)], col_b[b], sem_i[b])

        def drain_idx(g, b):
            off = off_of(g)
            pltpu.make_async_copy(row_h.at[pl.ds(off, KCH)], row_b[b], sem_i[b]).wait()
            pltpu.make_async_copy(col_h.at[pl.ds(off, KCH)], col_b[b], sem_i[b]).wait()

        def half(g, b):
            drain_idx(g, b)
            for j in range(KCH // L):
                sl = pl.ds(j * L, L)
                rr = row_b[b][sl]
                cc = col_b[b][sl]
                av = plsc.load_gather(an_v, [rr])
                bv = plsc.load_gather(be_v, [cc])
                e_b[b][sl] = jnp.exp(_leaky(av + bv, 0.2))
                sr_b[b][sl] = rr
                sc_b[b][sl] = cc
            pltpu.sync_copy(e_b[b], eexp_h.at[pl.ds(off_of(g), KCH)])
            pltpu.sync_copy(e_b[b], dnm_s.at[sc_b[b]], add=True)
            if with_counts:
                pltpu.sync_copy(one_v, b_s.at[sc_b[b]], add=True)
                pltpu.sync_copy(one_v, d_s.at[sr_b[b]], add=True)

        issue_idx(0, 0)

        def pair(t, _):
            ga = 2 * t
            issue_idx(ga + 1, 1)
            half(ga, 0)
            issue_idx(ga + 2, 0)
            half(ga + 1, 1)
            return 0
        lax.fori_loop(0, (EW // KCH) // 2, pair, 0)
        drain_idx(EW // KCH, 0)

        plsc.subcore_barrier()
        doff = pl.multiple_of(c * P + s * SL, SL)
        pltpu.sync_copy(dnm_s.at[pl.ds(soff, SL)], dnm_h.at[pl.ds(doff, SL)])
        if with_counts:
            pltpu.sync_copy(b_s.at[pl.ds(soff, SL)], bc_h.at[pl.ds(doff, SL)])
            pltpu.sync_copy(d_s.at[pl.ds(soff, SL)], dc_h.at[pl.ds(doff, SL)])

    return pl.kernel(body, out_type=out_type, mesh=_MESH, scratch_types=scratch,
                     compiler_params=_SC_PARAMS)


_k1_counts = _make_k1(True)
_k1_plain = _make_k1(False)


def _make_k2(full):
    n_out = 4 if full else 2
    out_type = [jax.ShapeDtypeStruct((P,), jnp.float32)] * n_out
    scratch = [pltpu.VMEM((SLW,), jnp.float32) for _ in range(3)]

    def body(*args):
        if full:
            (dnm_h, bc_h, dc_h, dnminv_h, ubd_h, binv_h, dinv_h, t0, t1, t2) = args
        else:
            (dnm_h, binv_in_h, dnminv_h, ubd_h, t0, t1, t2) = args
        wid = _wid()
        off = pl.multiple_of(wid * SLW, SLW)

        pltpu.sync_copy(dnm_h.at[pl.ds(off, SLW)], t0)
        pltpu.sync_copy(dnm_h.at[pl.ds(P + off, SLW)], t1)
        for j in range(SLW // L):
            sl = pl.ds(j * L, L)
            t0[sl] = 1.0 / (t0[sl] + t1[sl] + 1e-16)
        pltpu.sync_copy(t0, dnminv_h.at[pl.ds(off, SLW)])

        if full:
            pltpu.sync_copy(bc_h.at[pl.ds(off, SLW)], t1)
            pltpu.sync_copy(bc_h.at[pl.ds(P + off, SLW)], t2)
            for j in range(SLW // L):
                sl = pl.ds(j * L, L)
                b = t1[sl] + t2[sl]
                bi = jnp.where(b > 0, 1.0 / jnp.where(b > 0, b, 1.0), 0.0)
                t1[sl] = bi
                t2[sl] = bi * t0[sl]
            pltpu.sync_copy(t1, binv_h.at[pl.ds(off, SLW)])
            pltpu.sync_copy(t2, ubd_h.at[pl.ds(off, SLW)])

            pltpu.sync_copy(dc_h.at[pl.ds(off, SLW)], t1)
            pltpu.sync_copy(dc_h.at[pl.ds(P + off, SLW)], t2)
            for j in range(SLW // L):
                sl = pl.ds(j * L, L)
                d = t1[sl] + t2[sl]
                t1[sl] = jnp.where(d > 0, 1.0 / jnp.where(d > 0, d, 1.0), 0.0)
            pltpu.sync_copy(t1, dinv_h.at[pl.ds(off, SLW)])
        else:
            pltpu.sync_copy(binv_in_h.at[pl.ds(off, SLW)], t1)
            for j in range(SLW // L):
                sl = pl.ds(j * L, L)
                t2[sl] = t1[sl] * t0[sl]
            pltpu.sync_copy(t2, ubd_h.at[pl.ds(off, SLW)])

    return pl.kernel(body, out_type=out_type, mesh=_MESH, scratch_types=scratch,
                     compiler_params=_SC_PARAMS)


_k2_full = _make_k2(True)
_k2_small = _make_k2(False)


def _make_k3(stage1):
    out_type = [jax.ShapeDtypeStruct((NC * P, FEAT), jnp.float32)]
    if stage1:
        out_type.append(jax.ShapeDtypeStruct((EP2,), jnp.float32))
    scratch = [
        pltpu.VMEM((P,), jnp.float32),
        pltpu.VMEM((CH,), jnp.int32),
        pltpu.VMEM((CH,), jnp.int32),
        pltpu.VMEM((CH,), jnp.int32),
        pltpu.VMEM((CH,), jnp.int32),
        pltpu.VMEM((CH,), jnp.int32),
        pltpu.VMEM((CH,), jnp.int32),
        pltpu.VMEM((CH,), jnp.float32),
        pltpu.VMEM((CH,), jnp.float32),
        pltpu.VMEM((CH,), jnp.float32),
        pltpu.VMEM((CH,), jnp.float32),
        pltpu.VMEM((CH, FEAT), jnp.float32),
        pltpu.VMEM((CH, FEAT), jnp.float32),
        pltpu.VMEM_SHARED((P, FEAT), jnp.float32),
        pltpu.SemaphoreType.DMA,
        pltpu.SemaphoreType.DMA,
        pltpu.SemaphoreType.DMA,
        pltpu.SemaphoreType.DMA,
        pltpu.SemaphoreType.DMA,
        pltpu.SemaphoreType.DMA,
        pltpu.SemaphoreType.DMA,
    ]
    if stage1:
        scratch.insert(1, pltpu.VMEM((P,), jnp.float32))
        scratch[12:12] = [
            pltpu.VMEM((CH,), jnp.float32),
            pltpu.VMEM((CH,), jnp.float32),
        ]


    def body(row_h, col_h, e_h, *rest):
        if stage1:
            (ubd_h, dnminv_h, src_h, dst_h, enorm_h,
             u_v, dnm_v, rv0, rv1, cv0, cv1, si0, si1, ev0, ev1, wv0, wv1,
             en0, en1, rs0, rs1, dst_s,
             sem_i0, sem_i1, sem_g0, sem_g1, sem_s0, sem_s1, sem_e) = rest
        else:
            (dinv_h, src_h, dst_h,
             u_v, rv0, rv1, cv0, cv1, si0, si1, ev0, ev1, wv0, wv1,
             rs0, rs1, dst_s,
             sem_i0, sem_i1, sem_g0, sem_g1, sem_s0, sem_s1, sem_e) = rest
        c = lax.axis_index("c")
        s = lax.axis_index("s")
        wid = _wid()
        row_b = (rv0, rv1)
        col_b = (cv0, cv1)
        si_b = (si0, si1)
        e_b = (ev0, ev1)
        w_b = (wv0, wv1)
        if stage1:
            en_b = (en0, en1)
        rows_b = (rs0, rs1)
        sem_i = (sem_i0, sem_i1)
        sem_g = (sem_g0, sem_g1)
        sem_s = (sem_s0, sem_s1)

        def zrow(i, _):
            for j in range(FEAT // L):
                rs0[i, pl.ds(j * L, L)] = _zero16()
            return 0
        lax.fori_loop(0, CH, zrow, 0)
        soff = pl.multiple_of(s * SL, SL)
        for k in range(SL // CH):
            pltpu.sync_copy(rs0, dst_s.at[pl.ds(soff + k * CH, CH)])

        if stage1:
            pltpu.sync_copy(ubd_h, u_v)
            pltpu.sync_copy(dnminv_h, dnm_v)
        else:
            pltpu.sync_copy(dinv_h, u_v)
        plsc.subcore_barrier()

        base = pl.multiple_of(wid * EW, EW)

        def off_of(g):
            return pl.multiple_of(base + g * CH, CH)

        def issue_idx(g, b):
            off = off_of(g)
            pltpu.async_copy(row_h.at[pl.ds(off, CH)], row_b[b], sem_i[b])
            pltpu.async_copy(col_h.at[pl.ds(off, CH)], col_b[b], sem_i[b])
            pltpu.async_copy(e_h.at[pl.ds(off, CH)], e_b[b], sem_i[b])

        def drain_idx(g, b):
            off = off_of(g)
            pltpu.make_async_copy(row_h.at[pl.ds(off, CH)], row_b[b], sem_i[b]).wait()
            pltpu.make_async_copy(col_h.at[pl.ds(off, CH)], col_b[b], sem_i[b]).wait()
            pltpu.make_async_copy(e_h.at[pl.ds(off, CH)], e_b[b], sem_i[b]).wait()

        def gather_of(b):
            src_idx = row_b[b] if stage1 else col_b[b]
            return [pltpu.async_copy(src_h.at[src_idx], rows_b[b], sem_g[b])]

        def weights_of(g, b):
            dst_idx = col_b[b] if stage1 else row_b[b]
            for j in range(CH // L):
                sl = pl.ds(j * L, L)
                ii = dst_idx[sl]
                uu = plsc.load_gather(u_v, [ii])
                ee = e_b[b][sl]
                w_b[b][sl] = ee * uu
                si_b[b][sl] = ii
                if stage1:
                    dn = plsc.load_gather(dnm_v, [ii])
                    en_b[b][sl] = ee * dn
            if stage1:
                return pltpu.async_copy(
                    en_b[b], enorm_h.at[pl.ds(off_of(g), CH)], sem_e)
            return None

        def scale_of(b):
            def scale(b2, _):
                wvec = w_b[b][pl.ds(pl.multiple_of(b2 * L, L), L)]
                for i in range(L):
                    r = b2 * L + i
                    w = wvec[i]
                    for j in range(FEAT // L):
                        sl = pl.ds(j * L, L)
                        rows_b[b][r, sl] = rows_b[b][r, sl] * w
                return 0
            lax.fori_loop(0, CH // L, scale, 0)

        issue_idx(0, 0)

        def pair(t, _):
            ga = 2 * t
            issue_idx(ga + 1, 1)
            drain_idx(ga, 0)
            gcp0 = gather_of(0)
            ecp0 = weights_of(ga, 0)
            drain_idx(ga + 1, 1)
            gcp1 = gather_of(1)
            ecp1 = weights_of(ga + 1, 1)
            for cp in gcp0:
                cp.wait()
            issue_idx(ga + 2, 0)
            scale_of(0)
            scp0 = pltpu.async_copy(rows_b[0], dst_s.at[si_b[0]], sem_s[0], add=True)
            for cp in gcp1:
                cp.wait()
            scale_of(1)
            scp0.wait()
            scp1 = pltpu.async_copy(rows_b[1], dst_s.at[si_b[1]], sem_s[1], add=True)
            if stage1:
                ecp0.wait()
            scp1.wait()
            if stage1:
                ecp1.wait()
            return 0
        lax.fori_loop(0, GPW // 2, pair, 0)
        drain_idx(GPW, 0)

        plsc.subcore_barrier()
        for k in range(SL // CH):
            pltpu.sync_copy(
                dst_s.at[pl.ds(soff + k * CH, CH)],
                dst_h.at[pl.ds(pl.multiple_of(c * P + s * SL + k * CH, CH), CH)])

    return pl.kernel(body, out_type=out_type, mesh=_MESH, scratch_types=scratch,
                     compiler_params=_SC_PARAMS)


_k3_stage1 = _make_k3(True)
_k3_stage2 = _make_k3(False)


def kernel(x, edge_index, edge_attr, W1, att1, b1, W2, att2, b2,
           gn1_w, gn1_b, gn1_ms, gn2_w, gn2_b, gn2_ms,
           fc1_w, fc1_b, fc2_w, fc2_b, cls_w, cls_b):
    xp = jnp.pad(x, ((0, P - N), (0, 0)))
    eap = jnp.pad(edge_attr, ((0, P - M), (0, 0)))
    pad_idx = (jnp.arange(EP2 - E, dtype=jnp.int32) % (P - N)) + N
    rowp = jnp.concatenate([edge_index[0], pad_idx])
    colp = jnp.concatenate([edge_index[1], pad_idx])

    att1c = att1.reshape(2 * FEAT, 1)
    att2c = att2.reshape(2 * FEAT, 1)

    xl1, a1, b1v = _t1(xp, eap, W1, att1c)
    eexp1, dnm_p, bc_p, dc_p = _k1_counts(rowp, colp, a1.reshape(P), b1v.reshape(P))
    dnminv1, ubd1, binv, dinv = _k2_full(dnm_p, bc_p, dc_p)
    eout_p, enorm1 = _k3_stage1(rowp, colp, eexp1, ubd1, dnminv1, xl1)
    eout1 = _tadd(eout_p[:P], eout_p[P:])
    (out_p1,) = _k3_stage2(rowp, colp, enorm1, dinv, eout1)

    xl2, a2, b2v, ofc1 = _t2(
        out_p1[:P], out_p1[P:], b1.reshape(1, FEAT),
        gn1_w.reshape(1, FEAT), gn1_b.reshape(1, FEAT), gn1_ms.reshape(1, FEAT),
        fc1_w, fc1_b.reshape(1, HID), eap, W2, att2c)

    eexp2, dnm_p2 = _k1_plain(rowp, colp, a2.reshape(P), b2v.reshape(P))
    dnminv2, ubd2 = _k2_small(dnm_p2, binv)
    eout_p2, enorm2 = _k3_stage1(rowp, colp, eexp2, ubd2, dnminv2, xl2)
    eout2 = _tadd(eout_p2[:P], eout_p2[P:])
    (out_p2,) = _k3_stage2(rowp, colp, enorm2, dinv, eout2)

    res = _t3(
        out_p2[:P], out_p2[P:], b2.reshape(1, FEAT),
        gn2_w.reshape(1, FEAT), gn2_b.reshape(1, FEAT), gn2_ms.reshape(1, FEAT),
        fc2_w, fc2_b.reshape(1, HID), ofc1, cls_w, cls_b.reshape(1, OUT))
    return res[:N]


# speedup vs baseline: 1.5313x; 1.0037x over previous
import functools

import jax
import jax.numpy as jnp
from jax import lax
from jax.experimental import pallas as pl
from jax.experimental.pallas import tpu as pltpu
from jax.experimental.pallas import tpu_sc as plsc

N = 10000
M = 10000
E = 320000
FEAT = 128
HID = 64
OUT = 10

NC, NS, L = 2, 16, 16
NW = NC * NS
P = 10240
CH = 64
EW = 10240
GPW = EW // CH
EP = EW * NW
KCH = 128
EP2 = EP + KCH
SL = P // NS
SLW = P // NW

_MESH = plsc.VectorSubcoreMesh(core_axis_name="c", subcore_axis_name="s")
_SC_PARAMS = pltpu.CompilerParams(needs_layout_passes=False, use_tc_tiling_on_sc=False)


def _leaky(x, slope):
    return jnp.where(x > 0, x, slope * x)


def _wid():
    return lax.axis_index("s") * NC + lax.axis_index("c")


def _zero16():
    return jnp.zeros((L,), jnp.float32)


def _t1_body(x_ref, ea_ref, w_ref, att_ref, xl_ref, a_ref, b_ref):
    w = w_ref[...]
    xl = jnp.dot(x_ref[...], w, preferred_element_type=jnp.float32)
    el = jnp.dot(ea_ref[...], w, preferred_element_type=jnp.float32)
    xl_ref[...] = xl
    a_ref[...] = jnp.dot(xl, att_ref[0:FEAT, :], preferred_element_type=jnp.float32)
    b_ref[...] = jnp.dot(el, att_ref[FEAT:2 * FEAT, :], preferred_element_type=jnp.float32)


_t1 = pl.pallas_call(
    _t1_body,
    out_shape=[
        jax.ShapeDtypeStruct((P, FEAT), jnp.float32),
        jax.ShapeDtypeStruct((P, 1), jnp.float32),
        jax.ShapeDtypeStruct((P, 1), jnp.float32),
    ],
)


def _gnorm(y, gw, gb, gms, mask):
    cnt = jnp.float32(N)
    mean = jnp.sum(jnp.where(mask, y, 0.0), axis=0, keepdims=True) / cnt
    out = y - mean * gms
    om = jnp.where(mask, out, 0.0)
    var = jnp.sum(om * om, axis=0, keepdims=True) / cnt
    return gw * out / jnp.sqrt(var + 1e-5) + gb


def _t2_body(o0_ref, o1_ref, bias_ref, gw_ref, gb_ref, gms_ref, fw_ref, fb_ref,
             ea_ref, w2_ref, att_ref, xl_ref, a_ref, b_ref, ofc_ref):
    y = o0_ref[...] + o1_ref[...] + bias_ref[...]
    mask = lax.broadcasted_iota(jnp.int32, (P, FEAT), 0) < N
    h = _leaky(_gnorm(y, gw_ref[...], gb_ref[...], gms_ref[...], mask), 0.01)
    ofc_ref[...] = _leaky(
        jnp.dot(h, fw_ref[...], preferred_element_type=jnp.float32) + fb_ref[...], 0.01)
    w2 = w2_ref[...]
    xl = jnp.dot(h, w2, preferred_element_type=jnp.float32)
    el = jnp.dot(ea_ref[...], w2, preferred_element_type=jnp.float32)
    xl_ref[...] = xl
    a_ref[...] = jnp.dot(xl, att_ref[0:FEAT, :], preferred_element_type=jnp.float32)
    b_ref[...] = jnp.dot(el, att_ref[FEAT:2 * FEAT, :], preferred_element_type=jnp.float32)


_t2 = pl.pallas_call(
    _t2_body,
    out_shape=[
        jax.ShapeDtypeStruct((P, FEAT), jnp.float32),
        jax.ShapeDtypeStruct((P, 1), jnp.float32),
        jax.ShapeDtypeStruct((P, 1), jnp.float32),
        jax.ShapeDtypeStruct((P, HID), jnp.float32),
    ],
)


def _t3_body(o0_ref, o1_ref, bias_ref, gw_ref, gb_ref, gms_ref, fw_ref, fb_ref,
             ofc_ref, cw_ref, cb_ref, res_ref):
    y = o0_ref[...] + o1_ref[...] + bias_ref[...]
    mask = lax.broadcasted_iota(jnp.int32, (P, FEAT), 0) < N
    h2 = _leaky(_gnorm(y, gw_ref[...], gb_ref[...], gms_ref[...], mask), 0.01)
    out = ofc_ref[...] + _leaky(
        jnp.dot(h2, fw_ref[...], preferred_element_type=jnp.float32) + fb_ref[...], 0.01)
    res_ref[...] = jnp.dot(out, cw_ref[...], preferred_element_type=jnp.float32) + cb_ref[...]


_t3 = pl.pallas_call(
    _t3_body,
    out_shape=jax.ShapeDtypeStruct((P, OUT), jnp.float32),
)


def _tadd_body(a_ref, b_ref, o_ref):
    o_ref[...] = a_ref[...] + b_ref[...]


_tadd = pl.pallas_call(
    _tadd_body,
    out_shape=jax.ShapeDtypeStruct((P, FEAT), jnp.float32),
)


def _make_k1(with_counts):
    out_type = [jax.ShapeDtypeStruct((EP2,), jnp.float32),
                jax.ShapeDtypeStruct((NC * P,), jnp.float32)]
    scratch = [
        pltpu.VMEM((P,), jnp.float32),
        pltpu.VMEM((P,), jnp.float32),
        pltpu.VMEM((KCH,), jnp.int32),
        pltpu.VMEM((KCH,), jnp.int32),
        pltpu.VMEM((KCH,), jnp.int32),
        pltpu.VMEM((KCH,), jnp.int32),
        pltpu.VMEM((KCH,), jnp.int32),
        pltpu.VMEM((KCH,), jnp.int32),
        pltpu.VMEM((KCH,), jnp.int32),
        pltpu.VMEM((KCH,), jnp.int32),
        pltpu.VMEM((KCH,), jnp.float32),
        pltpu.VMEM((KCH,), jnp.float32),
        pltpu.VMEM((SL,), jnp.float32),
        pltpu.VMEM_SHARED((P,), jnp.float32),
        pltpu.SemaphoreType.DMA,
        pltpu.SemaphoreType.DMA,
        pltpu.SemaphoreType.DMA,
        pltpu.SemaphoreType.DMA,
    ]
    if with_counts:
        out_type += [jax.ShapeDtypeStruct((NC * P,), jnp.float32),
                     jax.ShapeDtypeStruct((NC * P,), jnp.float32)]
        scratch += [
            pltpu.VMEM((KCH,), jnp.float32),
            pltpu.VMEM_SHARED((P,), jnp.float32),
            pltpu.VMEM_SHARED((P,), jnp.float32),
        ]

    def body(row_h, col_h, an_h, be_h, *rest):
        if with_counts:
            (eexp_h, dnm_h, bc_h, dc_h,
             an_v, be_v, rv0, rv1, cv0, cv1, sr0, sr1, sc0, sc1, ev0, ev1,
             z_v, dnm_s, sem_i0, sem_i1, sem_s0, sem_s1, one_v, b_s, d_s) = rest
        else:
            (eexp_h, dnm_h,
             an_v, be_v, rv0, rv1, cv0, cv1, sr0, sr1, sc0, sc1, ev0, ev1,
             z_v, dnm_s, sem_i0, sem_i1, sem_s0, sem_s1) = rest
        c = lax.axis_index("c")
        s = lax.axis_index("s")
        wid = _wid()
        row_b = (rv0, rv1)
        col_b = (cv0, cv1)
        sr_b = (sr0, sr1)
        sc_b = (sc0, sc1)
        e_b = (ev0, ev1)
        sem_i = (sem_i0, sem_i1)
        sem_s = (sem_s0, sem_s1)

        def zb(i, _):
            z_v[pl.ds(pl.multiple_of(i * L, L), L)] = _zero16()
            return 0
        lax.fori_loop(0, SL // L, zb, 0)
        soff = pl.multiple_of(s * SL, SL)
        pltpu.sync_copy(z_v, dnm_s.at[pl.ds(soff, SL)])
        if with_counts:
            pltpu.sync_copy(z_v, b_s.at[pl.ds(soff, SL)])
            pltpu.sync_copy(z_v, d_s.at[pl.ds(soff, SL)])
            for j in range(KCH // L):
                one_v[pl.ds(j * L, L)] = jnp.full((L,), 1.0, jnp.float32)
        pltpu.sync_copy(an_h, an_v)
        pltpu.sync_copy(be_h, be_v)
        plsc.subcore_barrier()

        base = pl.multiple_of(wid * EW, EW)

        def off_of(g):
            return pl.multiple_of(base + g * KCH, KCH)

        def issue_idx(g, b):
            off = off_of(g)
            pltpu.async_copy(row_h.at[pl.ds(off, KCH)], row_b[b], sem_i[b])
            pltpu.async_copy(col_h.at[pl.ds(off, KCH)], col_b[b], sem_i[b])

        def drain_idx(g, b):
            off = off_of(g)
            pltpu.make_async_copy(row_h.at[pl.ds(off, KCH)], row_b[b], sem_i[b]).wait()
            pltpu.make_async_copy(col_h.at[pl.ds(off, KCH)], col_b[b], sem_i[b]).wait()

        def half(g, b):
            drain_idx(g, b)
            for j in range(KCH // L):
                sl = pl.ds(j * L, L)
                rr = row_b[b][sl]
                cc = col_b[b][sl]
                av = plsc.load_gather(an_v, [rr])
                bv = plsc.load_gather(be_v, [cc])
                e_b[b][sl] = jnp.exp(_leaky(av + bv, 0.2))
                sr_b[b][sl] = rr
                sc_b[b][sl] = cc
            pltpu.sync_copy(e_b[b], eexp_h.at[pl.ds(off_of(g), KCH)])
            pltpu.sync_copy(e_b[b], dnm_s.at[sc_b[b]], add=True)
            if with_counts:
                pltpu.sync_copy(one_v, b_s.at[sc_b[b]], add=True)
                pltpu.sync_copy(one_v, d_s.at[sr_b[b]], add=True)

        issue_idx(0, 0)

        def pair(t, _):
            ga = 2 * t
            issue_idx(ga + 1, 1)
            half(ga, 0)
            issue_idx(ga + 2, 0)
            half(ga + 1, 1)
            return 0
        lax.fori_loop(0, (EW // KCH) // 2, pair, 0)
        drain_idx(EW // KCH, 0)

        plsc.subcore_barrier()
        doff = pl.multiple_of(c * P + s * SL, SL)
        pltpu.sync_copy(dnm_s.at[pl.ds(soff, SL)], dnm_h.at[pl.ds(doff, SL)])
        if with_counts:
            pltpu.sync_copy(b_s.at[pl.ds(soff, SL)], bc_h.at[pl.ds(doff, SL)])
            pltpu.sync_copy(d_s.at[pl.ds(soff, SL)], dc_h.at[pl.ds(doff, SL)])

    return pl.kernel(body, out_type=out_type, mesh=_MESH, scratch_types=scratch,
                     compiler_params=_SC_PARAMS)


_k1_counts = _make_k1(True)
_k1_plain = _make_k1(False)


def _make_k2(full):
    n_out = 4 if full else 2
    out_type = [jax.ShapeDtypeStruct((P,), jnp.float32)] * n_out
    scratch = [pltpu.VMEM((SLW,), jnp.float32) for _ in range(3)]

    def body(*args):
        if full:
            (dnm_h, bc_h, dc_h, dnminv_h, ubd_h, binv_h, dinv_h, t0, t1, t2) = args
        else:
            (dnm_h, binv_in_h, dnminv_h, ubd_h, t0, t1, t2) = args
        wid = _wid()
        off = pl.multiple_of(wid * SLW, SLW)

        pltpu.sync_copy(dnm_h.at[pl.ds(off, SLW)], t0)
        pltpu.sync_copy(dnm_h.at[pl.ds(P + off, SLW)], t1)
        for j in range(SLW // L):
            sl = pl.ds(j * L, L)
            t0[sl] = 1.0 / (t0[sl] + t1[sl] + 1e-16)
        pltpu.sync_copy(t0, dnminv_h.at[pl.ds(off, SLW)])

        if full:
            pltpu.sync_copy(bc_h.at[pl.ds(off, SLW)], t1)
            pltpu.sync_copy(bc_h.at[pl.ds(P + off, SLW)], t2)
            for j in range(SLW // L):
                sl = pl.ds(j * L, L)
                b = t1[sl] + t2[sl]
                bi = jnp.where(b > 0, 1.0 / jnp.where(b > 0, b, 1.0), 0.0)
                t1[sl] = bi
                t2[sl] = bi * t0[sl]
            pltpu.sync_copy(t1, binv_h.at[pl.ds(off, SLW)])
            pltpu.sync_copy(t2, ubd_h.at[pl.ds(off, SLW)])

            pltpu.sync_copy(dc_h.at[pl.ds(off, SLW)], t1)
            pltpu.sync_copy(dc_h.at[pl.ds(P + off, SLW)], t2)
            for j in range(SLW // L):
                sl = pl.ds(j * L, L)
                d = t1[sl] + t2[sl]
                t1[sl] = jnp.where(d > 0, 1.0 / jnp.where(d > 0, d, 1.0), 0.0)
            pltpu.sync_copy(t1, dinv_h.at[pl.ds(off, SLW)])
        else:
            pltpu.sync_copy(binv_in_h.at[pl.ds(off, SLW)], t1)
            for j in range(SLW // L):
                sl = pl.ds(j * L, L)
                t2[sl] = t1[sl] * t0[sl]
            pltpu.sync_copy(t2, ubd_h.at[pl.ds(off, SLW)])

    return pl.kernel(body, out_type=out_type, mesh=_MESH, scratch_types=scratch,
                     compiler_params=_SC_PARAMS)


_k2_full = _make_k2(True)
_k2_small = _make_k2(False)


def _make_k3(stage1):
    out_type = [jax.ShapeDtypeStruct((NC * P, FEAT), jnp.float32)]
    if stage1:
        out_type.append(jax.ShapeDtypeStruct((EP2,), jnp.float32))
    scratch = [
        pltpu.VMEM((P,), jnp.float32),
        pltpu.VMEM((CH,), jnp.int32),
        pltpu.VMEM((CH,), jnp.int32),
        pltpu.VMEM((CH,), jnp.int32),
        pltpu.VMEM((CH,), jnp.int32),
        pltpu.VMEM((CH,), jnp.int32),
        pltpu.VMEM((CH,), jnp.int32),
        pltpu.VMEM((CH,), jnp.float32),
        pltpu.VMEM((CH,), jnp.float32),
        pltpu.VMEM((CH,), jnp.float32),
        pltpu.VMEM((CH,), jnp.float32),
        pltpu.VMEM((CH, FEAT), jnp.float32),
        pltpu.VMEM((CH, FEAT), jnp.float32),
        pltpu.VMEM_SHARED((P, FEAT), jnp.float32),
        pltpu.SemaphoreType.DMA,
        pltpu.SemaphoreType.DMA,
        pltpu.SemaphoreType.DMA,
        pltpu.SemaphoreType.DMA,
        pltpu.SemaphoreType.DMA,
        pltpu.SemaphoreType.DMA,
        pltpu.SemaphoreType.DMA,
    ]
    if stage1:
        scratch.insert(1, pltpu.VMEM((P,), jnp.float32))
        scratch[12:12] = [
            pltpu.VMEM((CH,), jnp.float32),
            pltpu.VMEM((CH,), jnp.float32),
        ]


    def body(row_h, col_h, e_h, *rest):
        if stage1:
            (ubd_h, dnminv_h, src_h, dst_h, enorm_h,
             u_v, dnm_v, rv0, rv1, cv0, cv1, si0, si1, ev0, ev1, wv0, wv1,
             en0, en1, rs0, rs1, dst_s,
             sem_i0, sem_i1, sem_g0, sem_g1, sem_s0, sem_s1, sem_e) = rest
        else:
            (dinv_h, src_h, dst_h,
             u_v, rv0, rv1, cv0, cv1, si0, si1, ev0, ev1, wv0, wv1,
             rs0, rs1, dst_s,
             sem_i0, sem_i1, sem_g0, sem_g1, sem_s0, sem_s1, sem_e) = rest
        c = lax.axis_index("c")
        s = lax.axis_index("s")
        wid = _wid()
        row_b = (rv0, rv1)
        col_b = (cv0, cv1)
        si_b = (si0, si1)
        e_b = (ev0, ev1)
        w_b = (wv0, wv1)
        if stage1:
            en_b = (en0, en1)
        rows_b = (rs0, rs1)
        sem_i = (sem_i0, sem_i1)
        sem_g = (sem_g0, sem_g1)
        sem_s = (sem_s0, sem_s1)

        def zrow(i, _):
            for j in range(FEAT // L):
                rs0[i, pl.ds(j * L, L)] = _zero16()
            return 0
        lax.fori_loop(0, CH, zrow, 0)
        soff = pl.multiple_of(s * SL, SL)
        for k in range(SL // CH):
            pltpu.sync_copy(rs0, dst_s.at[pl.ds(soff + k * CH, CH)])

        if stage1:
            pltpu.sync_copy(ubd_h, u_v)
            pltpu.sync_copy(dnminv_h, dnm_v)
        else:
            pltpu.sync_copy(dinv_h, u_v)
        plsc.subcore_barrier()

        base = pl.multiple_of(wid * EW, EW)

        def off_of(g):
            return pl.multiple_of(base + g * CH, CH)

        def issue_idx(g, b):
            off = off_of(g)
            pltpu.async_copy(row_h.at[pl.ds(off, CH)], row_b[b], sem_i[b])
            pltpu.async_copy(col_h.at[pl.ds(off, CH)], col_b[b], sem_i[b])
            pltpu.async_copy(e_h.at[pl.ds(off, CH)], e_b[b], sem_i[b])

        def drain_idx(g, b):
            off = off_of(g)
            pltpu.make_async_copy(row_h.at[pl.ds(off, CH)], row_b[b], sem_i[b]).wait()
            pltpu.make_async_copy(col_h.at[pl.ds(off, CH)], col_b[b], sem_i[b]).wait()
            pltpu.make_async_copy(e_h.at[pl.ds(off, CH)], e_b[b], sem_i[b]).wait()

        def gather_of(b):
            src_idx = row_b[b] if stage1 else col_b[b]
            return [pltpu.async_copy(src_h.at[src_idx], rows_b[b], sem_g[b])]

        def weights_of(g, b):
            dst_idx = col_b[b] if stage1 else row_b[b]
            for j in range(CH // L):
                sl = pl.ds(j * L, L)
                ii = dst_idx[sl]
                uu = plsc.load_gather(u_v, [ii])
                ee = e_b[b][sl]
                w_b[b][sl] = ee * uu
                si_b[b][sl] = ii
                if stage1:
                    dn = plsc.load_gather(dnm_v, [ii])
                    en_b[b][sl] = ee * dn
            if stage1:
                return pltpu.async_copy(
                    en_b[b], enorm_h.at[pl.ds(off_of(g), CH)], sem_e)
            return None

        def scale_of(b):
            def scale(b2, _):
                wvec = w_b[b][pl.ds(pl.multiple_of(b2 * L, L), L)]
                for i in range(L):
                    r = b2 * L + i
                    w = wvec[i]
                    for j in range(FEAT // L):
                        sl = pl.ds(j * L, L)
                        rows_b[b][r, sl] = rows_b[b][r, sl] * w
                return 0
            lax.fori_loop(0, CH // L, scale, 0)

        issue_idx(0, 0)

        def pair(t, _):
            ga = 2 * t
            issue_idx(ga + 1, 1)
            drain_idx(ga, 0)
            gcp0 = gather_of(0)
            ecp0 = weights_of(ga, 0)
            drain_idx(ga + 1, 1)
            gcp1 = gather_of(1)
            ecp1 = weights_of(ga + 1, 1)
            for cp in gcp0:
                cp.wait()
            issue_idx(ga + 2, 0)
            scale_of(0)
            scp0 = pltpu.async_copy(rows_b[0], dst_s.at[si_b[0]], sem_s[0], add=True)
            for cp in gcp1:
                cp.wait()
            scale_of(1)
            scp1 = pltpu.async_copy(rows_b[1], dst_s.at[si_b[1]], sem_s[1], add=True)
            scp0.wait()
            if stage1:
                ecp0.wait()
            scp1.wait()
            if stage1:
                ecp1.wait()
            return 0
        lax.fori_loop(0, GPW // 2, pair, 0)
        drain_idx(GPW, 0)

        plsc.subcore_barrier()
        for k in range(SL // CH):
            pltpu.sync_copy(
                dst_s.at[pl.ds(soff + k * CH, CH)],
                dst_h.at[pl.ds(pl.multiple_of(c * P + s * SL + k * CH, CH), CH)])

    return pl.kernel(body, out_type=out_type, mesh=_MESH, scratch_types=scratch,
                     compiler_params=_SC_PARAMS)


_k3_stage1 = _make_k3(True)
_k3_stage2 = _make_k3(False)


def kernel(x, edge_index, edge_attr, W1, att1, b1, W2, att2, b2,
           gn1_w, gn1_b, gn1_ms, gn2_w, gn2_b, gn2_ms,
           fc1_w, fc1_b, fc2_w, fc2_b, cls_w, cls_b):
    xp = jnp.pad(x, ((0, P - N), (0, 0)))
    eap = jnp.pad(edge_attr, ((0, P - M), (0, 0)))
    pad_idx = (jnp.arange(EP2 - E, dtype=jnp.int32) % (P - N)) + N
    rowp = jnp.concatenate([edge_index[0], pad_idx])
    colp = jnp.concatenate([edge_index[1], pad_idx])

    att1c = att1.reshape(2 * FEAT, 1)
    att2c = att2.reshape(2 * FEAT, 1)

    xl1, a1, b1v = _t1(xp, eap, W1, att1c)
    eexp1, dnm_p, bc_p, dc_p = _k1_counts(rowp, colp, a1.reshape(P), b1v.reshape(P))
    dnminv1, ubd1, binv, dinv = _k2_full(dnm_p, bc_p, dc_p)
    eout_p, enorm1 = _k3_stage1(rowp, colp, eexp1, ubd1, dnminv1, xl1)
    eout1 = _tadd(eout_p[:P], eout_p[P:])
    (out_p1,) = _k3_stage2(rowp, colp, enorm1, dinv, eout1)

    xl2, a2, b2v, ofc1 = _t2(
        out_p1[:P], out_p1[P:], b1.reshape(1, FEAT),
        gn1_w.reshape(1, FEAT), gn1_b.reshape(1, FEAT), gn1_ms.reshape(1, FEAT),
        fc1_w, fc1_b.reshape(1, HID), eap, W2, att2c)

    eexp2, dnm_p2 = _k1_plain(rowp, colp, a2.reshape(P), b2v.reshape(P))
    dnminv2, ubd2 = _k2_small(dnm_p2, binv)
    eout_p2, enorm2 = _k3_stage1(rowp, colp, eexp2, ubd2, dnminv2, xl2)
    eout2 = _tadd(eout_p2[:P], eout_p2[P:])
    (out_p2,) = _k3_stage2(rowp, colp, enorm2, dinv, eout2)

    res = _t3(
        out_p2[:P], out_p2[P:], b2.reshape(1, FEAT),
        gn2_w.reshape(1, FEAT), gn2_b.reshape(1, FEAT), gn2_ms.reshape(1, FEAT),
        fc2_w, fc2_b.reshape(1, HID), ofc1, cls_w, cls_b.reshape(1, OUT))
    return res[:N]
